# baseline (speedup 1.0000x reference)
import jax
import jax.numpy as jnp
from jax import lax
from jax.experimental import pallas as pl
from jax.experimental.pallas import tpu as pltpu
from jax.experimental.pallas import tpu_sc as plsc

_N = 10000
_D = 128
_E = 320000
_NC, _NS = 2, 16
_NW = _NC * _NS
_CHUNK = 128
_CPW = 80
_NPAD = _CPW * _CHUNK
_EPAD = _NW * _CPW * _CHUNK
_RPT = _NPAD // _NS
_HROWS = _EPAD // _CHUNK
_DEG_RPT = _HROWS // _NS

_sc_mesh = plsc.VectorSubcoreMesh(core_axis_name="c", subcore_axis_name="s")


def _deg_body(srcdg, dstm, zn, out, idx, ones_v, acc, dsem):
    c = lax.axis_index("c")
    s = lax.axis_index("s")
    pltpu.sync_copy(zn.at[pl.ds(s * _RPT, _RPT)], acc.at[pl.ds(s * _RPT, _RPT)])

    @pl.when(c == 0)
    def _():
        pltpu.sync_copy(srcdg.at[pl.ds(s * _DEG_RPT, _DEG_RPT), :], idx)

    @pl.when(c == 1)
    def _():
        pltpu.sync_copy(dstm.at[pl.ds(s * _DEG_RPT, _DEG_RPT), :], idx)

    for i in range(_CHUNK // 16):
        ones_v[pl.ds(i * 16, 16)] = jnp.full((16,), 1.0, jnp.float32)
    plsc.subcore_barrier()

    def group(g, carry):
        for b in range(8):
            pltpu.async_copy(ones_v, acc.at[idx.at[g * 8 + b]], dsem, add=True)
        for b in range(8):
            pltpu.make_async_copy(ones_v, acc.at[idx.at[g * 8 + b]],
                                  dsem).wait()
        return carry

    lax.fori_loop(0, _DEG_RPT // 8, group, 0)
    plsc.subcore_barrier()
    pltpu.sync_copy(acc.at[pl.ds(s * _RPT, _RPT)],
                    out.at[pl.ds(c * _NPAD + s * _RPT, _RPT)])


_deg_kernel = pl.kernel(
    _deg_body,
    out_type=jax.ShapeDtypeStruct((2 * _NPAD,), jnp.float32),
    mesh=_sc_mesh,
    scratch_types=[
        pltpu.VMEM((_DEG_RPT, _CHUNK), jnp.int32),
        pltpu.VMEM((_CHUNK,), jnp.float32),
        pltpu.VMEM_SHARED((_NPAD,), jnp.float32),
        pltpu.SemaphoreType.DMA,
    ],
)


_NBUF = 8
_LG = 3


def _mp_body(hs, srcm, dstm, zz, out0, out1, idx_s, idx_d, rows, acc,
             gsem, ssem):
    c = lax.axis_index("c")
    s = lax.axis_index("s")
    base = (c * _NS + s) * _CPW
    pltpu.sync_copy(zz.at[pl.ds(s * _RPT, _RPT), :],
                    acc.at[pl.ds(s * _RPT, _RPT), :])
    pltpu.sync_copy(srcm.at[pl.ds(base, _CPW), :], idx_s)
    pltpu.sync_copy(dstm.at[pl.ds(base, _CPW), :], idx_d)
    plsc.subcore_barrier()

    def g_issue(j, b):
        pltpu.async_copy(hs.at[idx_s.at[j]], rows.at[b], gsem.at[b])

    def g_wait(j, b):
        pltpu.make_async_copy(hs.at[idx_s.at[j]], rows.at[b], gsem.at[b]).wait()

    def s_issue(j, b):
        pltpu.async_copy(rows.at[b], acc.at[idx_d.at[j]], ssem.at[b], add=True)

    def s_wait(j, b):
        pltpu.make_async_copy(rows.at[b], acc.at[idx_d.at[j]],
                              ssem.at[b]).wait()

    for b in range(_NBUF):
        g_issue(b, b)
    for k in range(_NBUF - _LG):
        g_wait(k, k)
        s_issue(k, k)

    def group(g, carry):
        j0 = g * _NBUF
        for b in range(_NBUF):
            j = j0 + b
            s_wait(j - _NBUF, b)
            g_issue(j, b)
            b2 = (b + _NBUF - _LG) % _NBUF
            g_wait(j - _LG, b2)
            s_issue(j - _LG, b2)
        return carry

    lax.fori_loop(1, _CPW // _NBUF, group, 0)

    for j in range(_CPW - _LG, _CPW):
        g_wait(j, j % _NBUF)
        s_issue(j, j % _NBUF)
    for j in range(_CPW - _NBUF, _CPW):
        s_wait(j, j % _NBUF)
    plsc.subcore_barrier()

    @pl.when(c == 0)
    def _():
        pltpu.sync_copy(acc.at[pl.ds(s * _RPT, _RPT), :],
                        out0.at[pl.ds(s * _RPT, _RPT), :])

    @pl.when(c == 1)
    def _():
        pltpu.sync_copy(acc.at[pl.ds(s * _RPT, _RPT), :],
                        out1.at[pl.ds(s * _RPT, _RPT), :])


_mp_kernel = pl.kernel(
    _mp_body,
    out_type=[
        jax.ShapeDtypeStruct((_NPAD, _D), jnp.bfloat16),
        jax.ShapeDtypeStruct((_NPAD, _D), jnp.bfloat16),
    ],
    mesh=_sc_mesh,
    scratch_types=[
        pltpu.VMEM((_CPW, _CHUNK), jnp.int32),
        pltpu.VMEM((_CPW, _CHUNK), jnp.int32),
        pltpu.VMEM((_NBUF, _CHUNK, _D), jnp.bfloat16),
        pltpu.VMEM_SHARED((_NPAD, _D), jnp.bfloat16),
        pltpu.SemaphoreType.DMA((_NBUF,)),
        pltpu.SemaphoreType.DMA((_NBUF,)),
    ],
    compiler_params=pltpu.CompilerParams(use_tc_tiling_on_sc=False),
)

_RB = 2000


def _tc1_body(x, w, dg, out):
    norm = lax.rsqrt(jnp.maximum(dg[...], 1.0))
    r = jnp.dot(x[...], w[...], preferred_element_type=jnp.float32) * norm
    out[...] = r.astype(jnp.bfloat16)


_tc1 = pl.pallas_call(
    _tc1_body,
    grid=(_N // _RB,),
    in_specs=[
        pl.BlockSpec((_RB, _D), lambda i: (i, 0)),
        pl.BlockSpec((_D, _D), lambda i: (0, 0)),
        pl.BlockSpec((_RB, 1), lambda i: (i, 0)),
    ],
    out_specs=pl.BlockSpec((_RB, _D), lambda i: (i, 0)),
    out_shape=jax.ShapeDtypeStruct((_N, _D), jnp.bfloat16),
)


def _tc2_body(p0, p1, di, b, w, do, out):
    p = p0[...].astype(jnp.float32) + p1[...].astype(jnp.float32)
    m = p * lax.rsqrt(jnp.maximum(di[...], 1.0)) + b[...]
    h = jnp.maximum(m, 0.0)
    r = jnp.dot(h, w[...], preferred_element_type=jnp.float32) \
        * lax.rsqrt(jnp.maximum(do[...], 1.0))
    out[...] = r.astype(jnp.bfloat16)


_tc2 = pl.pallas_call(
    _tc2_body,
    grid=(_N // _RB,),
    in_specs=[
        pl.BlockSpec((_RB, _D), lambda i: (i, 0)),
        pl.BlockSpec((_RB, _D), lambda i: (i, 0)),
        pl.BlockSpec((_RB, 1), lambda i: (i, 0)),
        pl.BlockSpec((1, _D), lambda i: (0, 0)),
        pl.BlockSpec((_D, _D), lambda i: (0, 0)),
        pl.BlockSpec((_RB, 1), lambda i: (i, 0)),
    ],
    out_specs=pl.BlockSpec((_RB, _D), lambda i: (i, 0)),
    out_shape=jax.ShapeDtypeStruct((_N, _D), jnp.bfloat16),
)


def _tc3_body(p0, p1, di, b, out):
    p = p0[...].astype(jnp.float32) + p1[...].astype(jnp.float32)
    m = p * lax.rsqrt(jnp.maximum(di[...], 1.0)) + b[...]
    out[...] = jnp.maximum(m, 0.0)


_tc3 = pl.pallas_call(
    _tc3_body,
    grid=(_N // _RB,),
    in_specs=[
        pl.BlockSpec((_RB, _D), lambda i: (i, 0)),
        pl.BlockSpec((_RB, _D), lambda i: (i, 0)),
        pl.BlockSpec((_RB, 1), lambda i: (i, 0)),
        pl.BlockSpec((1, _D), lambda i: (0, 0)),
    ],
    out_specs=pl.BlockSpec((_RB, _D), lambda i: (i, 0)),
    out_shape=jax.ShapeDtypeStruct((_N, _D), jnp.float32),
)


def kernel(features, edge_index, W1, b1, W2, b2):
    src = edge_index[0]
    dst = edge_index[1]
    spread = jnp.arange(_EPAD - _E, dtype=jnp.int32) % (_NPAD - _N)
    src_m = jnp.concatenate([src, spread]).reshape(_HROWS, _CHUNK)
    src_dg = jnp.concatenate([src, _N + spread]).reshape(_HROWS, _CHUNK)
    dst_m = jnp.concatenate([dst, _N + spread]).reshape(_HROWS, _CHUNK)
    zn = jnp.zeros((_NPAD,), jnp.float32)
    znd = jnp.zeros((_NPAD, _D), jnp.bfloat16)

    degs = _deg_kernel(src_dg, dst_m, zn)
    deg_out = degs[:_N].reshape(_N, 1)
    deg_in = degs[_NPAD:_NPAD + _N].reshape(_N, 1)

    hs1 = _tc1(features, W1, deg_out)
    p0, p1 = _mp_kernel(hs1, src_m, dst_m, znd)

    hs2 = _tc2(p0, p1, deg_in, b1.reshape(1, _D), W2, deg_out)
    q0, q1 = _mp_kernel(hs2, src_m, dst_m, znd)

    return _tc3(q0, q1, deg_in, b2.reshape(1, _D))


# speedup vs baseline: 1.0327x; 1.0327x over previous
import jax
import jax.numpy as jnp
from jax import lax
from jax.experimental import pallas as pl
from jax.experimental.pallas import tpu as pltpu
from jax.experimental.pallas import tpu_sc as plsc

_N = 10000
_D = 128
_E = 320000
_NC, _NS = 2, 16
_NW = _NC * _NS
_CHUNK = 128
_CPW = 80
_NPAD = _CPW * _CHUNK
_EPAD = _NW * _CPW * _CHUNK
_RPT = _NPAD // _NS
_HROWS = _EPAD // _CHUNK
_DEG_RPT = 2 * _HROWS // _NW
_DH = _D // 2
_CPT = _HROWS // _NS

_sc_mesh = plsc.VectorSubcoreMesh(core_axis_name="c", subcore_axis_name="s")


def _deg_body(em, zn, out, idx, ones_v, acc, dsem):
    c = lax.axis_index("c")
    s = lax.axis_index("s")
    base = (c * _NS + s) * _DEG_RPT
    pltpu.sync_copy(zn.at[pl.ds(s * _RPT, _RPT)], acc.at[pl.ds(s * _RPT, _RPT)])
    pltpu.sync_copy(em.at[pl.ds(base, _DEG_RPT), :], idx)
    for i in range(_CHUNK // 16):
        ones_v[pl.ds(i * 16, 16)] = jnp.full((16,), 1.0, jnp.float32)
    plsc.subcore_barrier()

    def group(g, carry):
        for b in range(8):
            pltpu.async_copy(ones_v, acc.at[idx.at[g * 8 + b]], dsem, add=True)
        for b in range(8):
            pltpu.make_async_copy(ones_v, acc.at[idx.at[g * 8 + b]],
                                  dsem).wait()
        return carry

    lax.fori_loop(0, _DEG_RPT // 8, group, 0)
    plsc.subcore_barrier()
    pltpu.sync_copy(acc.at[pl.ds(s * _RPT, _RPT)],
                    out.at[pl.ds(c * _NPAD + s * _RPT, _RPT)])


_deg_kernel = pl.kernel(
    _deg_body,
    out_type=jax.ShapeDtypeStruct((2 * _NPAD,), jnp.float32),
    mesh=_sc_mesh,
    scratch_types=[
        pltpu.VMEM((_DEG_RPT, _CHUNK), jnp.int32),
        pltpu.VMEM((_CHUNK,), jnp.float32),
        pltpu.VMEM_SHARED((_NPAD,), jnp.float32),
        pltpu.SemaphoreType.DMA,
    ],
)


_NBUF = 8
_LG = 3


def _mp_body(hs, srcb, dstm, out, idx_s, idx_d, rows, acc, gsem, ssem):
    c = lax.axis_index("c")
    s = lax.axis_index("s")
    base = s * _CPT

    def zrow(i, carry):
        for k in range(_DH // 32):
            rows[0, i, pl.ds(k * 32, 32)] = jnp.zeros((32,), jnp.bfloat16)
        return carry

    lax.fori_loop(0, _CHUNK, zrow, 0)
    for r in range(_RPT // _CHUNK):
        pltpu.sync_copy(rows.at[0],
                        acc.at[pl.ds(s * _RPT + r * _CHUNK, _CHUNK), :])

    pltpu.sync_copy(srcb.at[pl.ds(c * _NS * _CPT + base, _CPT), :], idx_s)
    pltpu.sync_copy(dstm.at[pl.ds(base, _CPT), :], idx_d)
    plsc.subcore_barrier()

    def g_issue(j, b):
        pltpu.async_copy(hs.at[idx_s.at[j]], rows.at[b], gsem.at[b])

    def g_wait(j, b):
        pltpu.make_async_copy(hs.at[idx_s.at[j]], rows.at[b], gsem.at[b]).wait()

    def s_issue(j, b):
        pltpu.async_copy(rows.at[b], acc.at[idx_d.at[j]], ssem.at[b], add=True)

    def s_wait(j, b):
        pltpu.make_async_copy(rows.at[b], acc.at[idx_d.at[j]],
                              ssem.at[b]).wait()

    for b in range(_NBUF):
        g_issue(b, b)
    for k in range(_NBUF - _LG):
        g_wait(k, k)
        s_issue(k, k)

    def group(g, carry):
        j0 = g * _NBUF
        for b in range(_NBUF):
            j = j0 + b
            s_wait(j - _NBUF, b)
            g_issue(j, b)
            b2 = (b + _NBUF - _LG) % _NBUF
            g_wait(j - _LG, b2)
            s_issue(j - _LG, b2)
        return carry

    lax.fori_loop(1, _CPT // _NBUF, group, 0)

    for j in range(_CPT - _LG, _CPT):
        g_wait(j, j % _NBUF)
        s_issue(j, j % _NBUF)
    for j in range(_CPT - _NBUF, _CPT):
        s_wait(j, j % _NBUF)
    plsc.subcore_barrier()
    pltpu.sync_copy(acc.at[pl.ds(s * _RPT, _RPT), :],
                    out.at[pl.ds(s * _RPT, _RPT), pl.ds(c * _DH, _DH)])


_mp_kernel = pl.kernel(
    _mp_body,
    out_type=jax.ShapeDtypeStruct((_NPAD, _D), jnp.bfloat16),
    mesh=_sc_mesh,
    scratch_types=[
        pltpu.VMEM((_CPT, _CHUNK), jnp.int32),
        pltpu.VMEM((_CPT, _CHUNK), jnp.int32),
        pltpu.VMEM((_NBUF, _CHUNK, _DH), jnp.bfloat16),
        pltpu.VMEM_SHARED((_NPAD, _DH), jnp.bfloat16),
        pltpu.SemaphoreType.DMA((_NBUF,)),
        pltpu.SemaphoreType.DMA((_NBUF,)),
    ],
    compiler_params=pltpu.CompilerParams(use_tc_tiling_on_sc=False),
)

_RB = 2000


def _mm1_body(x, w, out):
    out[...] = jnp.dot(x[...], w[...], preferred_element_type=jnp.float32)


_mm1 = pl.pallas_call(
    _mm1_body,
    grid=(_N // _RB,),
    in_specs=[
        pl.BlockSpec((_RB, _D), lambda i: (i, 0)),
        pl.BlockSpec((_D, _D), lambda i: (0, 0)),
    ],
    out_specs=pl.BlockSpec((_RB, _D), lambda i: (i, 0)),
    out_shape=jax.ShapeDtypeStruct((_N, _D), jnp.float32),
)


def _scale1_body(y, dg, out):
    norm = lax.rsqrt(jnp.maximum(dg[...], 1.0))
    out[...] = (y[...] * norm).astype(jnp.bfloat16)


_scale1 = pl.pallas_call(
    _scale1_body,
    grid=(_N // _RB,),
    in_specs=[
        pl.BlockSpec((_RB, _D), lambda i: (i, 0)),
        pl.BlockSpec((_RB, 1), lambda i: (i, 0)),
    ],
    out_specs=pl.BlockSpec((_RB, _D), lambda i: (i, 0)),
    out_shape=jax.ShapeDtypeStruct((_N, _D), jnp.bfloat16),
)


def _tc2_body(p, di, b, w, do, out):
    m = p[...].astype(jnp.float32) * lax.rsqrt(jnp.maximum(di[...], 1.0)) \
        + b[...]
    h = jnp.maximum(m, 0.0)
    r = jnp.dot(h, w[...], preferred_element_type=jnp.float32) \
        * lax.rsqrt(jnp.maximum(do[...], 1.0))
    out[...] = r.astype(jnp.bfloat16)


_tc2 = pl.pallas_call(
    _tc2_body,
    grid=(_N // _RB,),
    in_specs=[
        pl.BlockSpec((_RB, _D), lambda i: (i, 0)),
        pl.BlockSpec((_RB, 1), lambda i: (i, 0)),
        pl.BlockSpec((1, _D), lambda i: (0, 0)),
        pl.BlockSpec((_D, _D), lambda i: (0, 0)),
        pl.BlockSpec((_RB, 1), lambda i: (i, 0)),
    ],
    out_specs=pl.BlockSpec((_RB, _D), lambda i: (i, 0)),
    out_shape=jax.ShapeDtypeStruct((_N, _D), jnp.bfloat16),
)


def _tc3_body(p, di, b, out):
    m = p[...].astype(jnp.float32) * lax.rsqrt(jnp.maximum(di[...], 1.0)) \
        + b[...]
    out[...] = jnp.maximum(m, 0.0)


_tc3 = pl.pallas_call(
    _tc3_body,
    grid=(_N // _RB,),
    in_specs=[
        pl.BlockSpec((_RB, _D), lambda i: (i, 0)),
        pl.BlockSpec((_RB, 1), lambda i: (i, 0)),
        pl.BlockSpec((1, _D), lambda i: (0, 0)),
    ],
    out_specs=pl.BlockSpec((_RB, _D), lambda i: (i, 0)),
    out_shape=jax.ShapeDtypeStruct((_N, _D), jnp.float32),
)


def kernel(features, edge_index, W1, b1, W2, b2):
    src = edge_index[0]
    dst = edge_index[1]
    spread = jnp.arange(_EPAD - _E, dtype=jnp.int32) % (_NPAD - _N)
    src_mp = jnp.concatenate([src, spread]).reshape(_HROWS, _CHUNK)
    src_dg = jnp.concatenate([src, _N + spread]).reshape(_HROWS, _CHUNK)
    dst_m = jnp.concatenate([dst, _N + spread]).reshape(_HROWS, _CHUNK)
    srcb = jnp.concatenate([2 * src_mp, 2 * src_mp + 1], axis=0)
    em = jnp.concatenate([src_dg, dst_m], axis=0)
    zn = jnp.zeros((_NPAD,), jnp.float32)

    degs = _deg_kernel(em, zn)
    deg_out = degs[:_N].reshape(_N, 1)
    deg_in = degs[_NPAD:_NPAD + _N].reshape(_N, 1)

    y1 = _mm1(features, W1)
    hs1 = _scale1(y1, deg_out).reshape(2 * _N, _DH)
    parts1 = _mp_kernel(hs1, srcb, dst_m)

    hs2 = _tc2(parts1[:_N], deg_in, b1.reshape(1, _D), W2,
               deg_out).reshape(2 * _N, _DH)
    parts2 = _mp_kernel(hs2, srcb, dst_m)

    return _tc3(parts2[:_N], deg_in, b2.reshape(1, _D))


# speedup vs baseline: 1.0359x; 1.0031x over previous
import jax
import jax.numpy as jnp
from jax import lax
from jax.experimental import pallas as pl
from jax.experimental.pallas import tpu as pltpu
from jax.experimental.pallas import tpu_sc as plsc

_N = 10000
_D = 128
_E = 320000
_NC, _NS = 2, 16
_NW = _NC * _NS
_CHUNK = 128
_CPW = 80
_NPAD = _CPW * _CHUNK
_EPAD = _NW * _CPW * _CHUNK
_RPT = _NPAD // _NS
_HROWS = _EPAD // _CHUNK
_DEG_RPT = 2 * _HROWS // _NW
_DH = _D // 2
_CPT = _HROWS // _NS

_sc_mesh = plsc.VectorSubcoreMesh(core_axis_name="c", subcore_axis_name="s")


def _deg_body(em, zn, out, idx, ones_v, acc, dsem):
    c = lax.axis_index("c")
    s = lax.axis_index("s")
    base = (c * _NS + s) * _DEG_RPT
    pltpu.sync_copy(zn.at[pl.ds(s * _RPT, _RPT)], acc.at[pl.ds(s * _RPT, _RPT)])
    pltpu.sync_copy(em.at[pl.ds(base, _DEG_RPT), :], idx)
    for i in range(_CHUNK // 16):
        ones_v[pl.ds(i * 16, 16)] = jnp.full((16,), 1.0, jnp.float32)
    plsc.subcore_barrier()

    def group(g, carry):
        for b in range(16):
            pltpu.async_copy(ones_v, acc.at[idx.at[g * 16 + b]], dsem,
                             add=True)
        for b in range(16):
            pltpu.make_async_copy(ones_v, acc.at[idx.at[g * 16 + b]],
                                  dsem).wait()
        return carry

    lax.fori_loop(0, _DEG_RPT // 16, group, 0)
    plsc.subcore_barrier()
    pltpu.sync_copy(acc.at[pl.ds(s * _RPT, _RPT)],
                    out.at[pl.ds(c * _NPAD + s * _RPT, _RPT)])


_deg_kernel = pl.kernel(
    _deg_body,
    out_type=jax.ShapeDtypeStruct((2 * _NPAD,), jnp.float32),
    mesh=_sc_mesh,
    scratch_types=[
        pltpu.VMEM((_DEG_RPT, _CHUNK), jnp.int32),
        pltpu.VMEM((_CHUNK,), jnp.float32),
        pltpu.VMEM_SHARED((_NPAD,), jnp.float32),
        pltpu.SemaphoreType.DMA,
    ],
)


_NBUF = 8
_LG = 3


def _mp_body(hs, srcb, dstm, out, idx_s, idx_d, rows, acc, gsem, ssem):
    c = lax.axis_index("c")
    s = lax.axis_index("s")
    base = s * _CPT

    def zrow(i, carry):
        for k in range(_DH // 32):
            rows[0, i, pl.ds(k * 32, 32)] = jnp.zeros((32,), jnp.bfloat16)
        return carry

    lax.fori_loop(0, _CHUNK, zrow, 0)
    for r in range(_RPT // _CHUNK):
        pltpu.sync_copy(rows.at[0],
                        acc.at[pl.ds(s * _RPT + r * _CHUNK, _CHUNK), :])

    pltpu.sync_copy(srcb.at[pl.ds(c * _NS * _CPT + base, _CPT), :], idx_s)
    pltpu.sync_copy(dstm.at[pl.ds(base, _CPT), :], idx_d)
    plsc.subcore_barrier()

    def g_issue(j, b):
        pltpu.async_copy(hs.at[idx_s.at[j]], rows.at[b], gsem.at[b])

    def g_wait(j, b):
        pltpu.make_async_copy(hs.at[idx_s.at[j]], rows.at[b], gsem.at[b]).wait()

    def s_issue(j, b):
        pltpu.async_copy(rows.at[b], acc.at[idx_d.at[j]], ssem.at[b], add=True)

    def s_wait(j, b):
        pltpu.make_async_copy(rows.at[b], acc.at[idx_d.at[j]],
                              ssem.at[b]).wait()

    for b in range(_NBUF):
        g_issue(b, b)
    for k in range(_NBUF - _LG):
        g_wait(k, k)
        s_issue(k, k)

    def group(g, carry):
        j0 = g * _NBUF
        for b in range(_NBUF):
            j = j0 + b
            s_wait(j - _NBUF, b)
            g_issue(j, b)
            b2 = (b + _NBUF - _LG) % _NBUF
            g_wait(j - _LG, b2)
            s_issue(j - _LG, b2)
        return carry

    lax.fori_loop(1, _CPT // _NBUF, group, 0)

    for j in range(_CPT - _LG, _CPT):
        g_wait(j, j % _NBUF)
        s_issue(j, j % _NBUF)
    for j in range(_CPT - _NBUF, _CPT):
        s_wait(j, j % _NBUF)
    plsc.subcore_barrier()
    pltpu.sync_copy(acc.at[pl.ds(s * _RPT, _RPT), :],
                    out.at[pl.ds(s * _RPT, _RPT), pl.ds(c * _DH, _DH)])


_mp_kernel = pl.kernel(
    _mp_body,
    out_type=jax.ShapeDtypeStruct((_NPAD, _D), jnp.bfloat16),
    mesh=_sc_mesh,
    scratch_types=[
        pltpu.VMEM((_CPT, _CHUNK), jnp.int32),
        pltpu.VMEM((_CPT, _CHUNK), jnp.int32),
        pltpu.VMEM((_NBUF, _CHUNK, _DH), jnp.bfloat16),
        pltpu.VMEM_SHARED((_NPAD, _DH), jnp.bfloat16),
        pltpu.SemaphoreType.DMA((_NBUF,)),
        pltpu.SemaphoreType.DMA((_NBUF,)),
    ],
    compiler_params=pltpu.CompilerParams(use_tc_tiling_on_sc=False),
)

_RB = 2000


def _mm1_body(x, w, out):
    out[...] = jnp.dot(x[...], w[...], preferred_element_type=jnp.float32)


_mm1 = pl.pallas_call(
    _mm1_body,
    grid=(_N // _RB,),
    in_specs=[
        pl.BlockSpec((_RB, _D), lambda i: (i, 0)),
        pl.BlockSpec((_D, _D), lambda i: (0, 0)),
    ],
    out_specs=pl.BlockSpec((_RB, _D), lambda i: (i, 0)),
    out_shape=jax.ShapeDtypeStruct((_N, _D), jnp.float32),
)


def _scale1_body(y, dg, out):
    norm = lax.rsqrt(jnp.maximum(dg[...], 1.0))
    out[...] = (y[...] * norm).astype(jnp.bfloat16)


_scale1 = pl.pallas_call(
    _scale1_body,
    grid=(_N // _RB,),
    in_specs=[
        pl.BlockSpec((_RB, _D), lambda i: (i, 0)),
        pl.BlockSpec((_RB, 1), lambda i: (i, 0)),
    ],
    out_specs=pl.BlockSpec((_RB, _D), lambda i: (i, 0)),
    out_shape=jax.ShapeDtypeStruct((_N, _D), jnp.bfloat16),
)


def _tc2_body(p, di, b, w, do, out):
    m = p[...].astype(jnp.float32) * lax.rsqrt(jnp.maximum(di[...], 1.0)) \
        + b[...]
    h = jnp.maximum(m, 0.0)
    r = jnp.dot(h, w[...], preferred_element_type=jnp.float32) \
        * lax.rsqrt(jnp.maximum(do[...], 1.0))
    out[...] = r.astype(jnp.bfloat16)


_tc2 = pl.pallas_call(
    _tc2_body,
    grid=(_N // _RB,),
    in_specs=[
        pl.BlockSpec((_RB, _D), lambda i: (i, 0)),
        pl.BlockSpec((_RB, 1), lambda i: (i, 0)),
        pl.BlockSpec((1, _D), lambda i: (0, 0)),
        pl.BlockSpec((_D, _D), lambda i: (0, 0)),
        pl.BlockSpec((_RB, 1), lambda i: (i, 0)),
    ],
    out_specs=pl.BlockSpec((_RB, _D), lambda i: (i, 0)),
    out_shape=jax.ShapeDtypeStruct((_N, _D), jnp.bfloat16),
)


def _tc3_body(p, di, b, out):
    m = p[...].astype(jnp.float32) * lax.rsqrt(jnp.maximum(di[...], 1.0)) \
        + b[...]
    out[...] = jnp.maximum(m, 0.0)


_tc3 = pl.pallas_call(
    _tc3_body,
    grid=(_N // _RB,),
    in_specs=[
        pl.BlockSpec((_RB, _D), lambda i: (i, 0)),
        pl.BlockSpec((_RB, 1), lambda i: (i, 0)),
        pl.BlockSpec((1, _D), lambda i: (0, 0)),
    ],
    out_specs=pl.BlockSpec((_RB, _D), lambda i: (i, 0)),
    out_shape=jax.ShapeDtypeStruct((_N, _D), jnp.float32),
)


def kernel(features, edge_index, W1, b1, W2, b2):
    src = edge_index[0]
    dst = edge_index[1]
    spread = jnp.arange(_EPAD - _E, dtype=jnp.int32) % (_NPAD - _N)
    src_mp = jnp.concatenate([src, spread]).reshape(_HROWS, _CHUNK)
    src_dg = jnp.concatenate([src, _N + spread]).reshape(_HROWS, _CHUNK)
    dst_m = jnp.concatenate([dst, _N + spread]).reshape(_HROWS, _CHUNK)
    srcb = jnp.concatenate([2 * src_mp, 2 * src_mp + 1], axis=0)
    em = jnp.concatenate([src_dg, dst_m], axis=0)
    zn = jnp.zeros((_NPAD,), jnp.float32)

    degs = _deg_kernel(em, zn)
    deg_out = degs[:_N].reshape(_N, 1)
    deg_in = degs[_NPAD:_NPAD + _N].reshape(_N, 1)

    y1 = _mm1(features, W1)
    hs1 = _scale1(y1, deg_out).reshape(2 * _N, _DH)
    parts1 = _mp_kernel(hs1, srcb, dst_m)

    hs2 = _tc2(parts1[:_N], deg_in, b1.reshape(1, _D), W2,
               deg_out).reshape(2 * _N, _DH)
    parts2 = _mp_kernel(hs2, srcb, dst_m)

    return _tc3(parts2[:_N], deg_in, b2.reshape(1, _D))
